# baseline (speedup 1.0000x reference)
import functools
import numpy as np
import jax
import jax.numpy as jnp
from jax import lax
from jax.experimental import pallas as pl
from jax.experimental.pallas import tpu as pltpu
from jax.experimental.pallas import tpu_sc as plsc

_ROT = ((13, 15, 26, 6), (17, 29, 16, 24))


def _np_threefry2x32(key, c1, c2):
    m = 0xFFFFFFFF
    ks = (key[0], key[1], key[0] ^ key[1] ^ 0x1BD11BDA)
    x0, x1 = (c1 + ks[0]) & m, (c2 + ks[1]) & m
    for i in range(5):
        for r in _ROT[i % 2]:
            x0 = (x0 + x1) & m
            x1 = ((x1 << r) | (x1 >> (32 - r))) & m
            x1 ^= x0
        x0 = (x0 + ks[(i + 1) % 3]) & m
        x1 = (x1 + ks[(i + 2) % 3] + i + 1) & m
    return x0, x1


_K_HI = _np_threefry2x32((0, 42), 0, 0)
_K_LO = _np_threefry2x32((0, 42), 0, 1)

_N = 4096 * 50
_NW = 32
_CHUNK = _N // _NW
_VEC = 16
_ITERS = _CHUNK // _VEC


def _cu(v):
    return jnp.full((_VEC,), np.uint32(v & 0xFFFFFFFF), jnp.uint32)


def _threefry_xor(key, cnt):
    ka, kb = key
    kc = ka ^ kb ^ 0x1BD11BDA
    ks = (ka, kb, kc)
    x0 = _cu(ka)
    x1 = cnt + _cu(kb)
    for i in range(5):
        for r in _ROT[i % 2]:
            x0 = x0 + x1
            x1 = (x1 << _cu(r)) | lax.shift_right_logical(x1, _cu(32 - r))
            x1 = x1 ^ x0
        x0 = x0 + _cu(ks[(i + 1) % 3])
        x1 = x1 + _cu(ks[(i + 2) % 3] + i + 1)
    return x0 ^ x1


def _mod99(n):
    t = lax.shift_right_logical(n, _cu(16)) * _cu(97) + (n & _cu(0xFFFF))
    t = lax.shift_right_logical(t, _cu(16)) * _cu(97) + (t & _cu(0xFFFF))
    q = lax.shift_right_logical(t * _cu(42367), _cu(22))
    return t - q * _cu(99)


_mesh = plsc.VectorSubcoreMesh(core_axis_name="c", subcore_axis_name="s")


@functools.partial(
    pl.kernel,
    mesh=_mesh,
    out_type=jax.ShapeDtypeStruct((_N,), jnp.int32),
    scratch_types=[
        pltpu.VMEM((_CHUNK,), jnp.int32),
        pltpu.VMEM((_CHUNK,), jnp.int32),
    ],
)
def _sc_kernel(x_hbm, out_hbm, x_v, o_v):
    wid = lax.axis_index("s") * 2 + lax.axis_index("c")
    base = wid * _CHUNK
    pltpu.sync_copy(x_hbm.at[pl.ds(base, _CHUNK)], x_v)
    iota = lax.iota(jnp.int32, _VEC)

    def body(j, carry):
        cnt_i = jnp.full((_VEC,), base + j * _VEC, jnp.int32) + iota
        cnt = lax.convert_element_type(cnt_i, jnp.uint32)
        hb = _threefry_xor(_K_HI, cnt)
        lb = _threefry_xor(_K_LO, cnt)
        v = _mod99(hb) * _cu(4) + _mod99(lb)
        q = lax.shift_right_logical(v * _cu(1325), _cu(17))
        off = lax.convert_element_type(v - q * _cu(99), jnp.int32)
        xv = x_v[pl.ds(j * _VEC, _VEC)]
        o_v[pl.ds(j * _VEC, _VEC)] = off + jnp.where(off >= xv, jnp.full((_VEC,), 1, jnp.int32), jnp.full((_VEC,), 0, jnp.int32))
        return carry

    lax.fori_loop(0, _ITERS, body, 0)
    pltpu.sync_copy(o_v, out_hbm.at[pl.ds(base, _CHUNK)])


def kernel(x):
    out = _sc_kernel(x.reshape(_N))
    return out.reshape(x.shape)


# speedup vs baseline: 2.4659x; 2.4659x over previous
import numpy as np
import jax
import jax.numpy as jnp
from jax import lax
from jax.experimental import pallas as pl
from jax.experimental.pallas import tpu as pltpu

_ROT = ((13, 15, 26, 6), (17, 29, 16, 24))


def _np_threefry2x32(key, c1, c2):
    m = 0xFFFFFFFF
    ks = (key[0], key[1], key[0] ^ key[1] ^ 0x1BD11BDA)
    x0, x1 = (c1 + ks[0]) & m, (c2 + ks[1]) & m
    for i in range(5):
        for r in _ROT[i % 2]:
            x0 = (x0 + x1) & m
            x1 = ((x1 << r) | (x1 >> (32 - r))) & m
            x1 ^= x0
        x0 = (x0 + ks[(i + 1) % 3]) & m
        x1 = (x1 + ks[(i + 2) % 3] + i + 1) & m
    return x0, x1


def _derived_keys():
    return _np_threefry2x32((0, 42), 0, 0), _np_threefry2x32((0, 42), 0, 1)


_K_HI, _K_LO = _derived_keys()

_N_ROWS = 1600
_LANES = 128
_GRID = 1
_BLK = _N_ROWS // _GRID


def _threefry_xor(key, cnt):
    ka, kb = key
    ks = (np.uint32(ka), np.uint32(kb), np.uint32(ka ^ kb ^ 0x1BD11BDA))
    x0 = jnp.full(cnt.shape, ks[0], jnp.uint32)
    x1 = cnt + ks[1]
    for i in range(5):
        for r in _ROT[i % 2]:
            x0 = x0 + x1
            x1 = (x1 << r) | lax.shift_right_logical(x1, np.uint32(32 - r))
            x1 = x1 ^ x0
        x0 = x0 + ks[(i + 1) % 3]
        x1 = x1 + np.uint32((int(ks[(i + 2) % 3]) + i + 1) & 0xFFFFFFFF)
    return x0 ^ x1


def _mod99(n):
    t = (n >> 16) * 97 + (n & 0xFFFF)
    t = (t >> 16) * 97 + (t & 0xFFFF)
    q = (t * 42367) >> 22
    return t - q * 99


def _body(x_ref, o_ref):
    g = pl.program_id(0)
    r = lax.broadcasted_iota(jnp.uint32, (_BLK, _LANES), 0)
    c = lax.broadcasted_iota(jnp.uint32, (_BLK, _LANES), 1)
    base = lax.convert_element_type(g, jnp.uint32) * np.uint32(_BLK)
    flat = (base + r) * np.uint32(_LANES) + c
    hb = _threefry_xor(_K_HI, flat)
    lb = _threefry_xor(_K_LO, flat)
    v = _mod99(hb) * 4 + _mod99(lb)
    q = (v * 1325) >> 17
    off = lax.convert_element_type(v - q * 99, jnp.int32)
    xv = x_ref[...]
    o_ref[...] = off + (off >= xv).astype(jnp.int32)


def kernel(x):
    xr = x.reshape(_N_ROWS, _LANES)
    out = pl.pallas_call(
        _body,
        out_shape=jax.ShapeDtypeStruct((_N_ROWS, _LANES), jnp.int32),
        grid=(_GRID,),
        in_specs=[pl.BlockSpec((_BLK, _LANES), lambda g: (g, 0))],
        out_specs=pl.BlockSpec((_BLK, _LANES), lambda g: (g, 0)),
        compiler_params=pltpu.CompilerParams(
            dimension_semantics=("parallel",)),
    )(xr)
    return out.reshape(x.shape)


# speedup vs baseline: 2.4875x; 1.0088x over previous
import numpy as np
import jax
import jax.numpy as jnp
from jax import lax
from jax.experimental import pallas as pl
from jax.experimental.pallas import tpu as pltpu

_ROT = ((13, 15, 26, 6), (17, 29, 16, 24))


def _np_threefry2x32(key, c1, c2):
    m = 0xFFFFFFFF
    ks = (key[0], key[1], key[0] ^ key[1] ^ 0x1BD11BDA)
    x0, x1 = (c1 + ks[0]) & m, (c2 + ks[1]) & m
    for i in range(5):
        for r in _ROT[i % 2]:
            x0 = (x0 + x1) & m
            x1 = ((x1 << r) | (x1 >> (32 - r))) & m
            x1 ^= x0
        x0 = (x0 + ks[(i + 1) % 3]) & m
        x1 = (x1 + ks[(i + 2) % 3] + i + 1) & m
    return x0, x1


def _derived_keys():
    return _np_threefry2x32((0, 42), 0, 0), _np_threefry2x32((0, 42), 0, 1)


_K_HI, _K_LO = _derived_keys()

_N_ROWS = 1600
_LANES = 128
_GRID = 2
_BLK = _N_ROWS // _GRID


def _threefry_xor(key, cnt):
    ka, kb = key
    ks = (np.uint32(ka), np.uint32(kb), np.uint32(ka ^ kb ^ 0x1BD11BDA))
    x0 = jnp.full(cnt.shape, ks[0], jnp.uint32)
    x1 = cnt + ks[1]
    for i in range(5):
        for r in _ROT[i % 2]:
            x0 = x0 + x1
            x1 = (x1 << r) | lax.shift_right_logical(x1, np.uint32(32 - r))
            x1 = x1 ^ x0
        x0 = x0 + ks[(i + 1) % 3]
        x1 = x1 + np.uint32((int(ks[(i + 2) % 3]) + i + 1) & 0xFFFFFFFF)
    return x0 ^ x1


def _mod99(n):
    t = (n >> 16) * 97 + (n & 0xFFFF)
    t = (t >> 16) * 97 + (t & 0xFFFF)
    q = (t * 42367) >> 22
    return t - q * 99


def _body(x_ref, o_ref):
    g = pl.program_id(0)
    r = lax.broadcasted_iota(jnp.uint32, (_BLK, _LANES), 0)
    c = lax.broadcasted_iota(jnp.uint32, (_BLK, _LANES), 1)
    base = lax.convert_element_type(g, jnp.uint32) * np.uint32(_BLK)
    flat = (base + r) * np.uint32(_LANES) + c
    hb = _threefry_xor(_K_HI, flat)
    lb = _threefry_xor(_K_LO, flat)
    v = _mod99(hb) * 4 + _mod99(lb)
    q = (v * 1325) >> 17
    off = lax.convert_element_type(v - q * 99, jnp.int32)
    xv = x_ref[...]
    o_ref[...] = off + (off >= xv).astype(jnp.int32)


def kernel(x):
    xr = x.reshape(_N_ROWS, _LANES)
    out = pl.pallas_call(
        _body,
        out_shape=jax.ShapeDtypeStruct((_N_ROWS, _LANES), jnp.int32),
        grid=(_GRID,),
        in_specs=[pl.BlockSpec((_BLK, _LANES), lambda g: (g, 0))],
        out_specs=pl.BlockSpec((_BLK, _LANES), lambda g: (g, 0)),
        compiler_params=pltpu.CompilerParams(
            dimension_semantics=("parallel",)),
    )(xr)
    return out.reshape(x.shape)


# speedup vs baseline: 2.4941x; 1.0027x over previous
import numpy as np
import jax
import jax.numpy as jnp
from jax import lax
from jax.experimental import pallas as pl
from jax.experimental.pallas import tpu as pltpu

_ROT = ((13, 15, 26, 6), (17, 29, 16, 24))


def _np_threefry2x32(key, c1, c2):
    m = 0xFFFFFFFF
    ks = (key[0], key[1], key[0] ^ key[1] ^ 0x1BD11BDA)
    x0, x1 = (c1 + ks[0]) & m, (c2 + ks[1]) & m
    for i in range(5):
        for r in _ROT[i % 2]:
            x0 = (x0 + x1) & m
            x1 = ((x1 << r) | (x1 >> (32 - r))) & m
            x1 ^= x0
        x0 = (x0 + ks[(i + 1) % 3]) & m
        x1 = (x1 + ks[(i + 2) % 3] + i + 1) & m
    return x0, x1


def _derived_keys():
    return _np_threefry2x32((0, 42), 0, 0), _np_threefry2x32((0, 42), 0, 1)


_K_HI, _K_LO = _derived_keys()

_N_ROWS = 1600
_LANES = 128
_GRID = 4
_BLK = _N_ROWS // _GRID


def _threefry_xor(key, cnt):
    ka, kb = key
    ks = (np.uint32(ka), np.uint32(kb), np.uint32(ka ^ kb ^ 0x1BD11BDA))
    x0 = jnp.full(cnt.shape, ks[0], jnp.uint32)
    x1 = cnt + ks[1]
    for i in range(5):
        for r in _ROT[i % 2]:
            x0 = x0 + x1
            x1 = (x1 << r) | lax.shift_right_logical(x1, np.uint32(32 - r))
            x1 = x1 ^ x0
        x0 = x0 + ks[(i + 1) % 3]
        x1 = x1 + np.uint32((int(ks[(i + 2) % 3]) + i + 1) & 0xFFFFFFFF)
    return x0 ^ x1


def _mod99(n):
    t = (n >> 16) * 97 + (n & 0xFFFF)
    t = (t >> 16) * 97 + (t & 0xFFFF)
    q = (t * 42367) >> 22
    return t - q * 99


def _body(x_ref, o_ref):
    g = pl.program_id(0)
    r = lax.broadcasted_iota(jnp.uint32, (_BLK, _LANES), 0)
    c = lax.broadcasted_iota(jnp.uint32, (_BLK, _LANES), 1)
    base = lax.convert_element_type(g, jnp.uint32) * np.uint32(_BLK)
    flat = (base + r) * np.uint32(_LANES) + c
    hb = _threefry_xor(_K_HI, flat)
    lb = _threefry_xor(_K_LO, flat)
    v = _mod99(hb) * 4 + _mod99(lb)
    q = (v * 1325) >> 17
    off = lax.convert_element_type(v - q * 99, jnp.int32)
    xv = x_ref[...]
    o_ref[...] = off + (off >= xv).astype(jnp.int32)


def kernel(x):
    xr = x.reshape(_N_ROWS, _LANES)
    out = pl.pallas_call(
        _body,
        out_shape=jax.ShapeDtypeStruct((_N_ROWS, _LANES), jnp.int32),
        grid=(_GRID,),
        in_specs=[pl.BlockSpec((_BLK, _LANES), lambda g: (g, 0))],
        out_specs=pl.BlockSpec((_BLK, _LANES), lambda g: (g, 0)),
        compiler_params=pltpu.CompilerParams(
            dimension_semantics=("parallel",)),
    )(xr)
    return out.reshape(x.shape)
